# baseline (speedup 1.0000x reference)
import functools

import jax
import jax.numpy as jnp
from jax import lax
from jax.experimental import pallas as pl
from jax.experimental.pallas import tpu as pltpu
from jax.experimental.pallas import tpu_sc as plsc

D = 768
E = 8
NSEG = D // 16
NC = 2
NS = 16
NW = NC * NS
CH = 128
B = 4


def _sc_body(x_hbm, w_hbm, o_hbm, xbuf, wbuf, obuf, sem):
    c = lax.axis_index("c")
    s = lax.axis_index("s")
    wid = s * NC + c
    tk = x_hbm.shape[0] // NW
    base = wid * tk

    pltpu.sync_copy(w_hbm, wbuf)

    def chunk_body(ci, _):
        rb = base + ci * CH
        pltpu.sync_copy(x_hbm.at[pl.ds(rb, CH)], xbuf)

        @plsc.parallel_loop(0, CH // B)
        def tb_body(bi):
            t0 = bi * B
            accs = [[jnp.zeros((16,), jnp.float32) for _ in range(E)]
                    for _ in range(B)]
            for j in range(NSEG):
                wv = [wbuf[e, pl.ds(j * 16, 16)] for e in range(E)]
                for t in range(B):
                    xv = xbuf[t0 + t, pl.ds(j * 16, 16)]
                    for e in range(E):
                        accs[t][e] = accs[t][e] + xv * wv[e]
            iota = lax.iota(jnp.int32, 16)
            for t in range(0, B, 2):
                out16 = jnp.zeros((16,), jnp.float32)
                for dt in (0, 1):
                    for e in range(E):
                        s = jnp.sum(accs[t + dt][e])
                        out16 = jnp.where(iota == (8 * dt + e), s, out16)
                off = pl.multiple_of((t0 + t) * E, 16)
                obuf[pl.ds(off, 16)] = out16

        pltpu.sync_copy(obuf, o_hbm.at[pl.ds(pl.multiple_of(rb * E, 8), CH * E)])
        return 0

    lax.fori_loop(0, tk // CH, chunk_body, 0)


def kernel(x, W):
    T = x.shape[0]
    mesh = plsc.VectorSubcoreMesh(core_axis_name="c", subcore_axis_name="s")
    k = functools.partial(
        pl.kernel,
        out_type=jax.ShapeDtypeStruct((T * E,), jnp.float32),
        mesh=mesh,
        compiler_params=pltpu.CompilerParams(needs_layout_passes=False),
        scratch_types=[
            pltpu.VMEM((CH, D), jnp.float32),
            pltpu.VMEM((E, D), jnp.float32),
            pltpu.VMEM((CH * E,), jnp.float32),
            pltpu.SemaphoreType.DMA,
        ],
    )(_sc_body)
    out = k(x, W)
    return out.reshape(T, E)
